# baseline (speedup 1.0000x reference)
import functools

import jax
import jax.numpy as jnp
from jax import lax
from jax.experimental import pallas as pl
from jax.experimental.pallas import tpu as pltpu
from jax.experimental.pallas import tpu_sc as plsc

_BATCH = 1024
_EMB = 16
_HID = 64
_NBR = 6
_NSLOT = 8
_XCOL = _NSLOT * _EMB
_NBLK = 1024

_NC = 2
_NS = 16
_NW = _NC * _NS

_NIDX = _NBR * _BATCH
_VPAD = 102400
_COLS = _VPAD // _NW
_MPAD = 1024
_MCOLS = _MPAD // _NW

_BR_CAP = 448
_MP_CAP = 128
_ZROWS = _BATCH // _NW
_SCH = 128
_NSC = 5
_ROWS = _NSC * _SCH
_ZBASE = _BR_CAP + _MP_CAP

_XROWS = _BATCH + 8


def _sc_gather_x(brt_t, mpt_t, br_idx, m_idx, br_pos, m_pos):
    mesh = plsc.VectorSubcoreMesh(core_axis_name="c", subcore_axis_name="s")

    @functools.partial(
        pl.kernel,
        mesh=mesh,
        out_type=jax.ShapeDtypeStruct((_XROWS * _NSLOT, _EMB), jnp.float32),
        scratch_types=[
            pltpu.VMEM((_EMB, _COLS // 128, 128), jnp.float32),
            pltpu.VMEM((_EMB, _MCOLS), jnp.float32),
            pltpu.VMEM((_NIDX,), jnp.int32),
            pltpu.VMEM((_BATCH,), jnp.int32),
            pltpu.VMEM((_NIDX,), jnp.int32),
            pltpu.VMEM((_BATCH,), jnp.int32),
            pltpu.VMEM((_BR_CAP + 16,), jnp.int32),
            pltpu.VMEM((_MP_CAP + 16,), jnp.int32),
            pltpu.VMEM((_NSC, _SCH), jnp.int32),
            pltpu.VMEM((_ROWS, _EMB), jnp.float32),
            pltpu.SemaphoreType.DMA,
            pltpu.SemaphoreType.DMA,
        ],
        compiler_params=pltpu.CompilerParams(use_tc_tiling_on_sc=False,
                                             needs_layout_passes=False),
    )
    def gather_kernel(brt_hbm, mpt_hbm, bidx_hbm, midx_hbm, bpos_hbm,
                      mpos_hbm, out_x, tloc, mloc, idx_v, midx_v, pos_v,
                      mpos_v, cbuf, mcbuf, pchunk, rows_v, sem, sem2):
        wid = lax.axis_index("s") * _NC + lax.axis_index("c")
        iota = lax.iota(jnp.int32, 16)
        zero16 = jnp.zeros((_EMB,), jnp.float32)
        dummy = jnp.full((16,), _BATCH * _NSLOT, jnp.int32) + wid

        cps = [
            pltpu.async_copy(brt_hbm.at[:, pl.ds(wid * (_COLS // 128),
                                                 _COLS // 128), :],
                             tloc, sem),
            pltpu.async_copy(mpt_hbm.at[:, wid // 4,
                                        pl.ds((wid % 4) * _MCOLS, _MCOLS)],
                             mloc, sem),
            pltpu.async_copy(bidx_hbm, idx_v, sem),
            pltpu.async_copy(midx_hbm, midx_v, sem),
            pltpu.async_copy(bpos_hbm, pos_v, sem),
            pltpu.async_copy(mpos_hbm, mpos_v, sem),
        ]

        for t in range(_BR_CAP // 16 + 1):
            cbuf[pl.ds(t * 16, 16)] = iota
        for t in range(_MP_CAP // 16 + 1):
            mcbuf[pl.ds(t * 16, 16)] = iota
        for ch in range(_NSC):
            for t in range(_SCH // 16):
                pchunk[ch, pl.ds(t * 16, 16)] = dummy

        for cp in cps:
            cp.wait()

        base = wid * _COLS
        mbase = wid * _MCOLS
        one16 = jnp.ones((16,), jnp.int32)
        z16 = jnp.zeros((16,), jnp.int32)

        def br_scan(i, off):
            jv = idx_v[pl.ds(i * 16, 16)]
            cols = jv - base
            mask = (cols >= 0) & (cols < _COLS)
            mi = jnp.where(mask, one16, z16)
            posv = off + plsc.cumsum(mi) - 1
            pv = pos_v[pl.ds(i * 16, 16)]
            plsc.store_scatter(cbuf, [posv], cols, mask=mask)
            plsc.store_scatter(pchunk, [posv >> 7, posv & 127], pv, mask=mask)
            return off + jnp.sum(mi)

        mbr = lax.fori_loop(0, _NIDX // 16, br_scan, 0)

        def mp_scan(i, off):
            jv = midx_v[pl.ds(i * 16, 16)]
            cols = jv - mbase
            mask = (cols >= 0) & (cols < _MCOLS)
            mi = jnp.where(mask, one16, z16)
            posv = off + plsc.cumsum(mi) - 1
            pv = mpos_v[pl.ds(i * 16, 16)]
            plsc.store_scatter(mcbuf, [posv - mbr], cols, mask=mask)
            plsc.store_scatter(pchunk, [posv >> 7, posv & 127], pv, mask=mask)
            return off + jnp.sum(mi)

        mtot = lax.fori_loop(0, _BATCH // 16, mp_scan, mbr) + _ZROWS

        def br_rows(t, _):
            cv = cbuf[pl.ds(t * 16, 16)]
            rowv = t * 16 + iota
            for c in range(_EMB):
                vals = plsc.load_gather(tloc, [jnp.full((16,), c, jnp.int32),
                                               cv >> 7, cv & 127])
                plsc.store_scatter(rows_v,
                                   [rowv, jnp.full((16,), c, jnp.int32)],
                                   vals)
            return 0

        lax.fori_loop(0, _BR_CAP // 16, br_rows, 0)

        def mp_rows(t, _):
            cv = mcbuf[pl.ds(t * 16, 16)]
            rowv = mbr + t * 16 + iota
            for c in range(_EMB):
                vals = plsc.load_gather(mloc, [jnp.full((16,), c, jnp.int32),
                                               cv])
                plsc.store_scatter(rows_v,
                                   [rowv, jnp.full((16,), c, jnp.int32)],
                                   vals)
            return 0

        lax.fori_loop(0, _MP_CAP // 16, mp_rows, 0)

        mz = mtot - _ZROWS
        for t in range(_ZROWS // 16):
            posz = mz + t * 16 + iota
            xrow = (wid * _ZROWS + t * 16 + iota) * _NSLOT + (_NSLOT - 1)
            plsc.store_scatter(pchunk, [posz >> 7, posz & 127], xrow)
        for r in range(_ZROWS):
            rows_v[mz + r, :] = zero16

        for ch in range(_NSC):
            @pl.when(ch * _SCH < mtot)
            def _():
                pltpu.async_copy(rows_v.at[pl.ds(ch * _SCH, _SCH)],
                                 out_x.at[pchunk.at[ch]], sem2).wait()

    return gather_kernel(brt_t, mpt_t, br_idx, m_idx, br_pos, m_pos)


def _mlp_body(x_ref, w1p_ref, b1_ref, w2_ref, b2_ref, out_ref, ht_ref):
    @pl.when(pl.program_id(0) == 0)
    def _():
        ht = lax.dot_general(w1p_ref[...], x_ref[...],
                             (((0,), (1,)), ((), ())),
                             preferred_element_type=jnp.float32)
        ht_ref[...] = jnp.maximum(ht + b1_ref[...], 0.0)

    out_ref[...] = (
        lax.dot_general(w2_ref[...], ht_ref[...],
                        (((0,), (0,)), ((), ())),
                        preferred_element_type=jnp.float32)
        + lax.dot_general(b2_ref[...], jnp.ones((1, _BATCH), jnp.float32),
                          (((0,), (0,)), ((), ())),
                          preferred_element_type=jnp.float32)
    )


def kernel(friends, enemies, map_idx, brawler_table, map_table, W1, b1, W2, b2):
    n_out = W2.shape[1]
    br_idx = jnp.concatenate([friends, enemies], axis=1).T.reshape(-1)
    m_idx = map_idx.reshape(-1)
    j = jnp.arange(_NIDX, dtype=jnp.int32)
    br_pos = (j % _BATCH) * _NSLOT + j // _BATCH
    m_pos = jnp.arange(_BATCH, dtype=jnp.int32) * _NSLOT + _NBR

    brt_t = jnp.pad(
        brawler_table.T,
        ((0, 0), (0, _VPAD - brawler_table.shape[0]))).reshape(
            _EMB, _VPAD // 128, 128)
    mpt_t = jnp.pad(
        map_table.T, ((0, 0), (0, _MPAD - map_table.shape[0]))).reshape(
            _EMB, _MPAD // 128, 128)

    x_rows = _sc_gather_x(brt_t, mpt_t, br_idx, m_idx, br_pos, m_pos)
    x = x_rows.reshape(_XROWS, _XCOL)

    w1p = jnp.pad(W1, ((0, _XCOL - W1.shape[0]), (0, 0)))
    b1c = b1.reshape(_HID, 1)
    b2r = b2.reshape(1, n_out)

    grid = pl.cdiv(n_out, _NBLK)
    out_t = pl.pallas_call(
        _mlp_body,
        grid=(grid,),
        in_specs=[
            pl.BlockSpec((_BATCH, _XCOL), lambda j: (0, 0)),
            pl.BlockSpec((_XCOL, _HID), lambda j: (0, 0)),
            pl.BlockSpec((_HID, 1), lambda j: (0, 0)),
            pl.BlockSpec((_HID, _NBLK), lambda j: (0, j)),
            pl.BlockSpec((1, _NBLK), lambda j: (0, j)),
        ],
        out_specs=pl.BlockSpec((_NBLK, _BATCH), lambda j: (j, 0)),
        out_shape=jax.ShapeDtypeStruct((n_out, _BATCH), jnp.float32),
        scratch_shapes=[pltpu.VMEM((_HID, _BATCH), jnp.float32)],
        compiler_params=pltpu.CompilerParams(
            dimension_semantics=("arbitrary",)),
    )(x, w1p, b1c, W2, b2r)
    return out_t.T


# speedup vs baseline: 1.1538x; 1.1538x over previous
import functools

import jax
import jax.numpy as jnp
from jax import lax
from jax.experimental import pallas as pl
from jax.experimental.pallas import tpu as pltpu
from jax.experimental.pallas import tpu_sc as plsc

_BATCH = 1024
_EMB = 16
_HID = 64
_NBR = 6
_NSLOT = 8
_XCOL = _NSLOT * _EMB
_NBLK = 5120

_NC = 2
_NS = 16
_NW = _NC * _NS

_NIDX = _NBR * _BATCH
_VPAD = 102400
_COLS = _VPAD // _NW
_MPAD = 1024
_MCOLS = _MPAD // _NW

_BR_CAP = 448
_MP_CAP = 128
_ZROWS = _BATCH // _NW
_SCH = 128
_NSC = 5
_ROWS = _NSC * _SCH
_ZBASE = _BR_CAP + _MP_CAP

_XROWS = _BATCH + 8


def _sc_gather_x(brt_t, mpt_t, br_idx, m_idx, br_pos, m_pos):
    mesh = plsc.VectorSubcoreMesh(core_axis_name="c", subcore_axis_name="s")

    @functools.partial(
        pl.kernel,
        mesh=mesh,
        out_type=jax.ShapeDtypeStruct((_XROWS * _NSLOT, _EMB), jnp.float32),
        scratch_types=[
            pltpu.VMEM((_EMB, _COLS // 128, 128), jnp.float32),
            pltpu.VMEM((_EMB, _MCOLS), jnp.float32),
            pltpu.VMEM((_NIDX,), jnp.int32),
            pltpu.VMEM((_BATCH,), jnp.int32),
            pltpu.VMEM((_NIDX,), jnp.int32),
            pltpu.VMEM((_BATCH,), jnp.int32),
            pltpu.VMEM((_BR_CAP + 16,), jnp.int32),
            pltpu.VMEM((_MP_CAP + 16,), jnp.int32),
            pltpu.VMEM((_NSC, _SCH), jnp.int32),
            pltpu.VMEM((_ROWS, _EMB), jnp.float32),
            pltpu.SemaphoreType.DMA,
            pltpu.SemaphoreType.DMA,
        ],
        compiler_params=pltpu.CompilerParams(use_tc_tiling_on_sc=False,
                                             needs_layout_passes=False),
    )
    def gather_kernel(brt_hbm, mpt_hbm, bidx_hbm, midx_hbm, bpos_hbm,
                      mpos_hbm, out_x, tloc, mloc, idx_v, midx_v, pos_v,
                      mpos_v, cbuf, mcbuf, pchunk, rows_v, sem, sem2):
        wid = lax.axis_index("s") * _NC + lax.axis_index("c")
        iota = lax.iota(jnp.int32, 16)
        zero16 = jnp.zeros((_EMB,), jnp.float32)
        dummy = jnp.full((16,), _BATCH * _NSLOT, jnp.int32) + wid

        cps = [
            pltpu.async_copy(brt_hbm.at[:, pl.ds(wid * (_COLS // 128),
                                                 _COLS // 128), :],
                             tloc, sem),
            pltpu.async_copy(mpt_hbm.at[:, wid // 4,
                                        pl.ds((wid % 4) * _MCOLS, _MCOLS)],
                             mloc, sem),
            pltpu.async_copy(bidx_hbm, idx_v, sem),
            pltpu.async_copy(midx_hbm, midx_v, sem),
            pltpu.async_copy(bpos_hbm, pos_v, sem),
            pltpu.async_copy(mpos_hbm, mpos_v, sem),
        ]

        for t in range(_BR_CAP // 16 + 1):
            cbuf[pl.ds(t * 16, 16)] = iota
        for t in range(_MP_CAP // 16 + 1):
            mcbuf[pl.ds(t * 16, 16)] = iota
        for ch in range(_NSC):
            for t in range(_SCH // 16):
                pchunk[ch, pl.ds(t * 16, 16)] = dummy

        for cp in cps:
            cp.wait()

        base = wid * _COLS
        mbase = wid * _MCOLS
        one16 = jnp.ones((16,), jnp.int32)
        z16 = jnp.zeros((16,), jnp.int32)

        def br_scan(i, off):
            jv = idx_v[pl.ds(i * 16, 16)]
            cols = jv - base
            mask = (cols >= 0) & (cols < _COLS)
            mi = jnp.where(mask, one16, z16)
            posv = off + plsc.cumsum(mi) - 1
            pv = pos_v[pl.ds(i * 16, 16)]
            plsc.store_scatter(cbuf, [posv], cols, mask=mask)
            plsc.store_scatter(pchunk, [posv >> 7, posv & 127], pv, mask=mask)
            return off + jnp.sum(mi)

        mbr = lax.fori_loop(0, _NIDX // 16, br_scan, 0)

        def mp_scan(i, off):
            jv = midx_v[pl.ds(i * 16, 16)]
            cols = jv - mbase
            mask = (cols >= 0) & (cols < _MCOLS)
            mi = jnp.where(mask, one16, z16)
            posv = off + plsc.cumsum(mi) - 1
            pv = mpos_v[pl.ds(i * 16, 16)]
            plsc.store_scatter(mcbuf, [posv - mbr], cols, mask=mask)
            plsc.store_scatter(pchunk, [posv >> 7, posv & 127], pv, mask=mask)
            return off + jnp.sum(mi)

        mtot = lax.fori_loop(0, _BATCH // 16, mp_scan, mbr) + _ZROWS

        def br_rows(t, _):
            cv = cbuf[pl.ds(t * 16, 16)]
            rowv = t * 16 + iota
            for c in range(_EMB):
                vals = plsc.load_gather(tloc, [jnp.full((16,), c, jnp.int32),
                                               cv >> 7, cv & 127])
                plsc.store_scatter(rows_v,
                                   [rowv, jnp.full((16,), c, jnp.int32)],
                                   vals)
            return 0

        lax.fori_loop(0, _BR_CAP // 16, br_rows, 0)

        def mp_rows(t, _):
            cv = mcbuf[pl.ds(t * 16, 16)]
            rowv = mbr + t * 16 + iota
            for c in range(_EMB):
                vals = plsc.load_gather(mloc, [jnp.full((16,), c, jnp.int32),
                                               cv])
                plsc.store_scatter(rows_v,
                                   [rowv, jnp.full((16,), c, jnp.int32)],
                                   vals)
            return 0

        lax.fori_loop(0, _MP_CAP // 16, mp_rows, 0)

        mz = mtot - _ZROWS
        for t in range(_ZROWS // 16):
            posz = mz + t * 16 + iota
            xrow = (wid * _ZROWS + t * 16 + iota) * _NSLOT + (_NSLOT - 1)
            plsc.store_scatter(pchunk, [posz >> 7, posz & 127], xrow)
        for r in range(_ZROWS):
            rows_v[mz + r, :] = zero16

        for ch in range(_NSC):
            @pl.when(ch * _SCH < mtot)
            def _():
                pltpu.async_copy(rows_v.at[pl.ds(ch * _SCH, _SCH)],
                                 out_x.at[pchunk.at[ch]], sem2).wait()

    return gather_kernel(brt_t, mpt_t, br_idx, m_idx, br_pos, m_pos)


def _mlp_body(x_ref, w1p_ref, b1_ref, w2_ref, b2_ref, out_ref, ht_ref):
    @pl.when(pl.program_id(0) == 0)
    def _():
        ht = lax.dot_general(w1p_ref[...], x_ref[...],
                             (((0,), (1,)), ((), ())),
                             preferred_element_type=jnp.float32)
        ht_ref[...] = jnp.maximum(ht + b1_ref[...], 0.0)

    out_ref[...] = (
        lax.dot_general(w2_ref[...], ht_ref[...],
                        (((0,), (0,)), ((), ())),
                        preferred_element_type=jnp.float32)
        + lax.dot_general(b2_ref[...], jnp.ones((1, _BATCH), jnp.float32),
                          (((0,), (0,)), ((), ())),
                          preferred_element_type=jnp.float32)
    )


def kernel(friends, enemies, map_idx, brawler_table, map_table, W1, b1, W2, b2):
    n_out = W2.shape[1]
    br_idx = jnp.concatenate([friends, enemies], axis=1).T.reshape(-1)
    m_idx = map_idx.reshape(-1)
    j = jnp.arange(_NIDX, dtype=jnp.int32)
    br_pos = (j % _BATCH) * _NSLOT + j // _BATCH
    m_pos = jnp.arange(_BATCH, dtype=jnp.int32) * _NSLOT + _NBR

    brt_t = jnp.pad(
        brawler_table.T,
        ((0, 0), (0, _VPAD - brawler_table.shape[0]))).reshape(
            _EMB, _VPAD // 128, 128)
    mpt_t = jnp.pad(
        map_table.T, ((0, 0), (0, _MPAD - map_table.shape[0]))).reshape(
            _EMB, _MPAD // 128, 128)

    x_rows = _sc_gather_x(brt_t, mpt_t, br_idx, m_idx, br_pos, m_pos)
    x = x_rows.reshape(_XROWS, _XCOL)

    w1p = jnp.pad(W1, ((0, _XCOL - W1.shape[0]), (0, 0)))
    b1c = b1.reshape(_HID, 1)
    b2r = b2.reshape(1, n_out)

    grid = pl.cdiv(n_out, _NBLK)
    out_t = pl.pallas_call(
        _mlp_body,
        grid=(grid,),
        in_specs=[
            pl.BlockSpec((_BATCH, _XCOL), lambda j: (0, 0)),
            pl.BlockSpec((_XCOL, _HID), lambda j: (0, 0)),
            pl.BlockSpec((_HID, 1), lambda j: (0, 0)),
            pl.BlockSpec((_HID, _NBLK), lambda j: (0, j)),
            pl.BlockSpec((1, _NBLK), lambda j: (0, j)),
        ],
        out_specs=pl.BlockSpec((_NBLK, _BATCH), lambda j: (j, 0)),
        out_shape=jax.ShapeDtypeStruct((n_out, _BATCH), jnp.float32),
        scratch_shapes=[pltpu.VMEM((_HID, _BATCH), jnp.float32)],
        compiler_params=pltpu.CompilerParams(
            dimension_semantics=("arbitrary",)),
    )(x, w1p, b1c, W2, b2r)
    return out_t.T


# speedup vs baseline: 1.2093x; 1.0481x over previous
import functools

import jax
import jax.numpy as jnp
from jax import lax
from jax.experimental import pallas as pl
from jax.experimental.pallas import tpu as pltpu
from jax.experimental.pallas import tpu_sc as plsc

_BATCH = 1024
_EMB = 16
_HID = 64
_NBR = 6
_NSLOT = 8
_XCOL = _NSLOT * _EMB
_NBLK = 4096

_NC = 2
_NS = 16
_NW = _NC * _NS

_NIDX = _NBR * _BATCH
_VPAD = 102400
_COLS = _VPAD // _NW
_MPAD = 1024
_MCOLS = _MPAD // _NW

_BR_CAP = 448
_MP_CAP = 128
_ZROWS = _BATCH // _NW
_SCH = 128
_NSC = 5
_ROWS = _NSC * _SCH

_XROWS = _BATCH + 8


def _sc_gather_x(brt_t, mpt_t, br_idx, m_idx, br_pos, m_pos):
    mesh = plsc.VectorSubcoreMesh(core_axis_name="c", subcore_axis_name="s")

    @functools.partial(
        pl.kernel,
        mesh=mesh,
        out_type=jax.ShapeDtypeStruct((_XROWS * _NSLOT, _EMB), jnp.float32),
        scratch_types=[
            pltpu.VMEM((2, _COLS // 128, 8, 128), jnp.float32),
            pltpu.VMEM((2, 8, _MCOLS), jnp.float32),
            pltpu.VMEM((_NIDX,), jnp.int32),
            pltpu.VMEM((_BATCH,), jnp.int32),
            pltpu.VMEM((_NIDX,), jnp.int32),
            pltpu.VMEM((_BATCH,), jnp.int32),
            pltpu.VMEM((_BR_CAP + 16,), jnp.int32),
            pltpu.VMEM((_MP_CAP + 16,), jnp.int32),
            pltpu.VMEM((_NSC, _SCH), jnp.int32),
            pltpu.VMEM((_ROWS, _EMB), jnp.float32),
            pltpu.SemaphoreType.DMA,
            pltpu.SemaphoreType.DMA,
        ],
        compiler_params=pltpu.CompilerParams(use_tc_tiling_on_sc=False,
                                             needs_layout_passes=False),
    )
    def gather_kernel(brt_hbm, mpt_hbm, bidx_hbm, midx_hbm, bpos_hbm,
                      mpos_hbm, out_x, tloc, mloc, idx_v, midx_v, pos_v,
                      mpos_v, cbuf, mcbuf, pchunk, rows_v, sem, sem2):
        wid = lax.axis_index("s") * _NC + lax.axis_index("c")
        iota = lax.iota(jnp.int32, 16)
        zero16 = jnp.zeros((_EMB,), jnp.float32)
        dummy = jnp.full((16,), _BATCH * _NSLOT, jnp.int32) + wid

        tbl_cps = [
            pltpu.async_copy(brt_hbm.at[:, pl.ds(wid * (_COLS // 128),
                                                 _COLS // 128), :, :],
                             tloc, sem2),
            pltpu.async_copy(mpt_hbm.at[:, wid // 4, :,
                                        pl.ds((wid % 4) * _MCOLS, _MCOLS)],
                             mloc, sem2),
        ]
        cps = [
            pltpu.async_copy(bidx_hbm, idx_v, sem),
            pltpu.async_copy(midx_hbm, midx_v, sem),
            pltpu.async_copy(bpos_hbm, pos_v, sem),
            pltpu.async_copy(mpos_hbm, mpos_v, sem),
        ]

        for t in range(_BR_CAP // 16 + 1):
            cbuf[pl.ds(t * 16, 16)] = iota
        for t in range(_MP_CAP // 16 + 1):
            mcbuf[pl.ds(t * 16, 16)] = iota
        for ch in range(_NSC):
            for t in range(_SCH // 16):
                pchunk[ch, pl.ds(t * 16, 16)] = dummy

        for cp in cps:
            cp.wait()

        base = wid * _COLS
        mbase = wid * _MCOLS
        one16 = jnp.ones((16,), jnp.int32)
        z16 = jnp.zeros((16,), jnp.int32)

        def br_scan(i, off):
            jv = idx_v[pl.ds(i * 16, 16)]
            cols = jv - base
            mask = (cols >= 0) & (cols < _COLS)
            mi = jnp.where(mask, one16, z16)
            posv = off + plsc.cumsum(mi) - 1
            pv = pos_v[pl.ds(i * 16, 16)]
            plsc.store_scatter(cbuf, [posv], cols, mask=mask)
            plsc.store_scatter(pchunk, [posv >> 7, posv & 127], pv, mask=mask)
            return off + jnp.sum(mi)

        mbr = lax.fori_loop(0, _NIDX // 16, br_scan, 0, unroll=4)

        def mp_scan(i, off):
            jv = midx_v[pl.ds(i * 16, 16)]
            cols = jv - mbase
            mask = (cols >= 0) & (cols < _MCOLS)
            mi = jnp.where(mask, one16, z16)
            posv = off + plsc.cumsum(mi) - 1
            pv = mpos_v[pl.ds(i * 16, 16)]
            plsc.store_scatter(mcbuf, [posv - mbr], cols, mask=mask)
            plsc.store_scatter(pchunk, [posv >> 7, posv & 127], pv, mask=mask)
            return off + jnp.sum(mi)

        mtot = lax.fori_loop(0, _BATCH // 16, mp_scan, mbr, unroll=4) + _ZROWS

        def br_rows(t, _):
            cv = cbuf[pl.ds(t * 16, 16)]
            rowv = t * 16 + iota
            for c in range(_EMB):
                vals = plsc.load_gather(
                    tloc, [jnp.full((16,), c >> 3, jnp.int32), cv >> 7,
                           jnp.full((16,), c & 7, jnp.int32), cv & 127])
                plsc.store_scatter(rows_v,
                                   [rowv, jnp.full((16,), c, jnp.int32)],
                                   vals)
            return 0

        lax.fori_loop(0, (mbr + 15) // 16, br_rows, 0)

        def mp_rows(t, _):
            cv = mcbuf[pl.ds(t * 16, 16)]
            rowv = mbr + t * 16 + iota
            for c in range(_EMB):
                vals = plsc.load_gather(
                    mloc, [jnp.full((16,), c >> 3, jnp.int32),
                           jnp.full((16,), c & 7, jnp.int32), cv])
                plsc.store_scatter(rows_v,
                                   [rowv, jnp.full((16,), c, jnp.int32)],
                                   vals)
            return 0

        lax.fori_loop(0, (mtot - _ZROWS - mbr + 15) // 16, mp_rows, 0)

        mz = mtot - _ZROWS
        for t in range(_ZROWS // 16):
            posz = mz + t * 16 + iota
            xrow = (wid * _ZROWS + t * 16 + iota) * _NSLOT + (_NSLOT - 1)
            plsc.store_scatter(pchunk, [posz >> 7, posz & 127], xrow)
        for r in range(_ZROWS):
            rows_v[mz + r, :] = zero16

        for ch in range(_NSC):
            @pl.when(ch * _SCH < mtot)
            def _():
                pltpu.async_copy(rows_v.at[pl.ds(ch * _SCH, _SCH)],
                                 out_x.at[pchunk.at[ch]], sem2).wait()

    return gather_kernel(brt_t, mpt_t, br_idx, m_idx, br_pos, m_pos)


def _mlp_body(x_ref, w1p_ref, b1_ref, w2_ref, b2_ref, out_ref, ht_ref):
    @pl.when(pl.program_id(0) == 0)
    def _():
        ht = lax.dot_general(w1p_ref[...], x_ref[...],
                             (((0,), (1,)), ((), ())),
                             preferred_element_type=jnp.float32)
        ht_ref[...] = jnp.maximum(ht + b1_ref[...], 0.0)

    out_ref[...] = (
        lax.dot_general(w2_ref[...], ht_ref[...],
                        (((0,), (0,)), ((), ())),
                        preferred_element_type=jnp.float32)
        + lax.dot_general(b2_ref[...], jnp.ones((1, _BATCH), jnp.float32),
                          (((0,), (0,)), ((), ())),
                          preferred_element_type=jnp.float32)
    )


def kernel(friends, enemies, map_idx, brawler_table, map_table, W1, b1, W2, b2):
    n_out = W2.shape[1]
    br_idx = jnp.concatenate([friends, enemies], axis=1).T.reshape(-1)
    m_idx = map_idx.reshape(-1)
    j = jnp.arange(_NIDX, dtype=jnp.int32)
    br_pos = (j % _BATCH) * _NSLOT + j // _BATCH
    m_pos = jnp.arange(_BATCH, dtype=jnp.int32) * _NSLOT + _NBR

    brt_t = jnp.pad(
        brawler_table.T,
        ((0, 0), (0, _VPAD - brawler_table.shape[0]))).reshape(
            2, 8, _VPAD // 128, 128).transpose(0, 2, 1, 3)
    mpt_t = jnp.pad(
        map_table.T, ((0, 0), (0, _MPAD - map_table.shape[0]))).reshape(
            2, 8, _MPAD // 128, 128).transpose(0, 2, 1, 3)

    x_rows = _sc_gather_x(brt_t, mpt_t, br_idx, m_idx, br_pos, m_pos)
    x = x_rows.reshape(_XROWS, _XCOL)

    w1p = jnp.pad(W1, ((0, _XCOL - W1.shape[0]), (0, 0)))
    b1c = b1.reshape(_HID, 1)
    b2r = b2.reshape(1, n_out)

    grid = pl.cdiv(n_out, _NBLK)
    out_t = pl.pallas_call(
        _mlp_body,
        grid=(grid,),
        in_specs=[
            pl.BlockSpec((_BATCH, _XCOL), lambda j: (0, 0)),
            pl.BlockSpec((_XCOL, _HID), lambda j: (0, 0)),
            pl.BlockSpec((_HID, 1), lambda j: (0, 0)),
            pl.BlockSpec((_HID, _NBLK), lambda j: (0, j)),
            pl.BlockSpec((1, _NBLK), lambda j: (0, j)),
        ],
        out_specs=pl.BlockSpec((_NBLK, _BATCH), lambda j: (j, 0)),
        out_shape=jax.ShapeDtypeStruct((n_out, _BATCH), jnp.float32),
        scratch_shapes=[pltpu.VMEM((_HID, _BATCH), jnp.float32)],
        compiler_params=pltpu.CompilerParams(
            dimension_semantics=("arbitrary",)),
    )(x, w1p, b1c, W2, b2r)
    return out_t.T
